# initial kernel scaffold (unmeasured)
import jax
import jax.numpy as jnp
from jax import lax
from jax.experimental import pallas as pl
from jax.experimental.pallas import tpu as pltpu

N_DEV = 8
SQ = 512
D = 1024
HQ = 8
DH = 128
SCALE = 0.08838834764831843
CW = D + 128
M_OFF = D
L_OFF = D + 8


def kernel(x, Wq, Wo, K_ext, V_ext):
    skv = K_ext.shape[1]
    x2 = x.reshape(SQ, D)
    k2 = K_ext.reshape(skv, HQ * DH)
    v2 = V_ext.reshape(skv, HQ * DH)

    def body(x_ref, wq_ref, wo_ref, k_ref, v_ref, out_ref,
             q_ref, comm_ref, acc_ref, send_sems, recv_sems):
        my = lax.axis_index("i")
        left = lax.rem(my + N_DEV - 1, N_DEV)
        right = lax.rem(my + 1, N_DEV)

        barrier_sem = pltpu.get_barrier_semaphore()
        for nbr in (left, right):
            pl.semaphore_signal(
                barrier_sem, inc=1,
                device_id=(nbr,), device_id_type=pl.DeviceIdType.MESH,
            )
        pl.semaphore_wait(barrier_sem, 2)

        q_ref[...] = jnp.dot(
            x_ref[...], wq_ref[...], preferred_element_type=jnp.float32
        )

        for h in range(HQ):
            c = slice(h * DH, (h + 1) * DH)
            s = lax.dot_general(
                q_ref[:, c], k_ref[:, c],
                (((1,), (1,)), ((), ())),
                preferred_element_type=jnp.float32,
            ) * SCALE
            m = jnp.max(s, axis=1, keepdims=True)
            p = jnp.exp(s - m)
            l = jnp.sum(p, axis=1, keepdims=True)
            u = jnp.dot(p, v_ref[:, c], preferred_element_type=jnp.float32)
            comm_ref[0, :, c] = u
            comm_ref[0, :, M_OFF + h:M_OFF + h + 1] = m
            comm_ref[0, :, L_OFF + h:L_OFF + h + 1] = l

        acc_ref[...] = comm_ref[0]

        def merge(slot):
            for h in range(HQ):
                c = slice(h * DH, (h + 1) * DH)
                ma = acc_ref[:, M_OFF + h:M_OFF + h + 1]
                mb = comm_ref[slot, :, M_OFF + h:M_OFF + h + 1]
                m_new = jnp.maximum(ma, mb)
                alpha = jnp.exp(ma - m_new)
                beta = jnp.exp(mb - m_new)
                acc_ref[:, c] = (
                    acc_ref[:, c] * alpha + comm_ref[slot, :, c] * beta
                )
                acc_ref[:, L_OFF + h:L_OFF + h + 1] = (
                    acc_ref[:, L_OFF + h:L_OFF + h + 1] * alpha
                    + comm_ref[slot, :, L_OFF + h:L_OFF + h + 1] * beta
                )
                acc_ref[:, M_OFF + h:M_OFF + h + 1] = m_new

        for h in range(N_DEV - 1):
            rdma = pltpu.make_async_remote_copy(
                src_ref=comm_ref.at[h],
                dst_ref=comm_ref.at[h + 1],
                send_sem=send_sems.at[h],
                recv_sem=recv_sems.at[h],
                device_id=(right,),
                device_id_type=pl.DeviceIdType.MESH,
            )
            rdma.start()
            rdma.wait()
            merge(h + 1)

        for h in range(HQ):
            c = slice(h * DH, (h + 1) * DH)
            acc_ref[:, c] = (
                acc_ref[:, c] / acc_ref[:, L_OFF + h:L_OFF + h + 1]
            )
        out_ref[...] = jnp.dot(
            acc_ref[:, :D], wo_ref[...], preferred_element_type=jnp.float32
        )

    out = pl.pallas_call(
        body,
        out_shape=jax.ShapeDtypeStruct((SQ, D), jnp.float32),
        in_specs=[pl.BlockSpec(memory_space=pltpu.VMEM)] * 5,
        out_specs=pl.BlockSpec(memory_space=pltpu.VMEM),
        scratch_shapes=[
            pltpu.VMEM((SQ, D), jnp.float32),
            pltpu.VMEM((N_DEV, SQ, CW), jnp.float32),
            pltpu.VMEM((SQ, CW), jnp.float32),
            pltpu.SemaphoreType.DMA((N_DEV - 1,)),
            pltpu.SemaphoreType.DMA((N_DEV - 1,)),
        ],
        compiler_params=pltpu.CompilerParams(collective_id=0),
    )(x2, Wq, Wo, k2, v2)
    return out.reshape(1, SQ, D)


# baseline (device time: 254853 ns/iter reference)
import jax
import jax.numpy as jnp
from jax import lax
from jax.experimental import pallas as pl
from jax.experimental.pallas import tpu as pltpu

N_DEV = 8
SQ = 512
D = 1024
HQ = 8
DH = 128
KV_CHUNK = 1024
SCALE = 0.08838834764831843
CW = D + 128
M_OFF = D
L_OFF = D + 8


def kernel(x, Wq, Wo, K_ext, V_ext):
    skv = K_ext.shape[1]
    x2 = x.reshape(SQ, D)
    k2 = K_ext.reshape(skv, HQ * DH)
    v2 = V_ext.reshape(skv, HQ * DH)

    def body(x_ref, wq_ref, wo_ref, k_ref, v_ref, out_ref,
             q_ref, comm_ref, acc_ref, send_sems, recv_sems, credit_sem):
        my = lax.axis_index("i")
        left = lax.rem(my + N_DEV - 1, N_DEV)
        right = lax.rem(my + 1, N_DEV)

        barrier_sem = pltpu.get_barrier_semaphore()
        for nbr in (left, right):
            pl.semaphore_signal(
                barrier_sem, inc=1,
                device_id=(nbr,), device_id_type=pl.DeviceIdType.MESH,
            )
        pl.semaphore_wait(barrier_sem, 2)

        q_ref[...] = jnp.dot(
            x_ref[...], wq_ref[...], preferred_element_type=jnp.float32
        )

        n_chunks = skv // KV_CHUNK
        for h in range(HQ):
            c = slice(h * DH, (h + 1) * DH)
            qh = q_ref[:, c]

            def chunk_body(ci, carry, c=c, qh=qh):
                m, l, u = carry
                r = pl.ds(ci * KV_CHUNK, KV_CHUNK)
                s = lax.dot_general(
                    qh, k_ref[r, c],
                    (((1,), (1,)), ((), ())),
                    preferred_element_type=jnp.float32,
                ) * SCALE
                mj = jnp.max(s, axis=1, keepdims=True)
                m_new = jnp.maximum(m, mj)
                alpha = jnp.exp(m - m_new)
                p = jnp.exp(s - m_new)
                l = l * alpha + jnp.sum(p, axis=1, keepdims=True)
                u = u * alpha + jnp.dot(
                    p, v_ref[r, c], preferred_element_type=jnp.float32)
                return m_new, l, u

            m0 = jnp.full((SQ, 1), -jnp.inf, dtype=jnp.float32)
            l0 = jnp.zeros((SQ, 1), dtype=jnp.float32)
            u0 = jnp.zeros((SQ, DH), dtype=jnp.float32)
            m, l, u = lax.fori_loop(0, n_chunks, chunk_body, (m0, l0, u0))
            comm_ref[0, :, c] = u
            comm_ref[0, :, M_OFF + h:M_OFF + h + 1] = m
            comm_ref[0, :, L_OFF + h:L_OFF + h + 1] = l

        acc_ref[...] = comm_ref[0]

        def merge(slot):
            for h in range(HQ):
                c = slice(h * DH, (h + 1) * DH)
                ma = acc_ref[:, M_OFF + h:M_OFF + h + 1]
                mb = comm_ref[slot, :, M_OFF + h:M_OFF + h + 1]
                m_new = jnp.maximum(ma, mb)
                alpha = jnp.exp(ma - m_new)
                beta = jnp.exp(mb - m_new)
                acc_ref[:, c] = (
                    acc_ref[:, c] * alpha + comm_ref[slot, :, c] * beta
                )
                acc_ref[:, L_OFF + h:L_OFF + h + 1] = (
                    acc_ref[:, L_OFF + h:L_OFF + h + 1] * alpha
                    + comm_ref[slot, :, L_OFF + h:L_OFF + h + 1] * beta
                )
                acc_ref[:, M_OFF + h:M_OFF + h + 1] = m_new

        for h in range(N_DEV - 1):
            if h >= 2:
                pl.semaphore_wait(credit_sem, 1)
            send_slot = h % 2
            recv_slot = (h + 1) % 2
            rdma = pltpu.make_async_remote_copy(
                src_ref=comm_ref.at[send_slot],
                dst_ref=comm_ref.at[recv_slot],
                send_sem=send_sems.at[send_slot],
                recv_sem=recv_sems.at[recv_slot],
                device_id=(right,),
                device_id_type=pl.DeviceIdType.MESH,
            )
            rdma.start()
            rdma.wait()
            if 1 <= h <= 5:
                pl.semaphore_signal(
                    credit_sem, inc=1,
                    device_id=(left,), device_id_type=pl.DeviceIdType.MESH,
                )
            merge(recv_slot)

        for h in range(HQ):
            c = slice(h * DH, (h + 1) * DH)
            acc_ref[:, c] = (
                acc_ref[:, c] / acc_ref[:, L_OFF + h:L_OFF + h + 1]
            )
        out_ref[...] = jnp.dot(
            acc_ref[:, :D], wo_ref[...], preferred_element_type=jnp.float32
        )

    out = pl.pallas_call(
        body,
        out_shape=jax.ShapeDtypeStruct((SQ, D), jnp.float32),
        in_specs=[pl.BlockSpec(memory_space=pltpu.VMEM)] * 5,
        out_specs=pl.BlockSpec(memory_space=pltpu.VMEM),
        scratch_shapes=[
            pltpu.VMEM((SQ, D), jnp.float32),
            pltpu.VMEM((2, SQ, CW), jnp.float32),
            pltpu.VMEM((SQ, CW), jnp.float32),
            pltpu.SemaphoreType.DMA((2,)),
            pltpu.SemaphoreType.DMA((2,)),
            pltpu.SemaphoreType.REGULAR,
        ],
        compiler_params=pltpu.CompilerParams(collective_id=0),
    )(x2, Wq, Wo, k2, v2)
    return out.reshape(1, SQ, D)


# device time: 116754 ns/iter; 2.1828x vs baseline; 2.1828x over previous
import jax
import jax.numpy as jnp
from jax import lax
from jax.experimental import pallas as pl
from jax.experimental.pallas import tpu as pltpu

N_DEV = 8
SQ = 512
D = 1024
HQ = 8
DH = 128
R = SQ // N_DEV
KV_CHUNK = 1024
SCALE = 0.08838834764831843
CW = D + 128
M_OFF = D
L_OFF = D + 8


def kernel(x, Wq, Wo, K_ext, V_ext):
    skv = K_ext.shape[1]
    x2 = x.reshape(SQ, D)
    k2 = K_ext.reshape(skv, HQ * DH)
    v2 = V_ext.reshape(skv, HQ * DH)

    def body(x_ref, wq_ref, wo_ref, k_ref, v_ref, out_ref,
             q_ref, acc_ref, rs_buf,
             rs_send_sems, rs_recv_sems, ag_send_sems, ag_recv_sems):
        my = lax.axis_index("i")
        left = lax.rem(my + N_DEV - 1, N_DEV)
        right = lax.rem(my + 1, N_DEV)

        barrier_sem = pltpu.get_barrier_semaphore()
        for nbr in (left, right):
            pl.semaphore_signal(
                barrier_sem, inc=1,
                device_id=(nbr,), device_id_type=pl.DeviceIdType.MESH,
            )
        pl.semaphore_wait(barrier_sem, 2)

        q_ref[...] = jnp.dot(
            x_ref[...], wq_ref[...], preferred_element_type=jnp.float32
        )

        n_chunks = skv // KV_CHUNK
        for h in range(HQ):
            c = slice(h * DH, (h + 1) * DH)
            qh = q_ref[:, c]

            def chunk_body(ci, carry, c=c, qh=qh):
                m, l, u = carry
                r = pl.ds(ci * KV_CHUNK, KV_CHUNK)
                s = lax.dot_general(
                    qh, k_ref[r, c],
                    (((1,), (1,)), ((), ())),
                    preferred_element_type=jnp.float32,
                ) * SCALE
                mj = jnp.max(s, axis=1, keepdims=True)
                m_new = jnp.maximum(m, mj)
                alpha = jnp.exp(m - m_new)
                p = jnp.exp(s - m_new)
                l = l * alpha + jnp.sum(p, axis=1, keepdims=True)
                u = u * alpha + jnp.dot(
                    p, v_ref[r, c], preferred_element_type=jnp.float32)
                return m_new, l, u

            m0 = jnp.full((SQ, 1), -jnp.inf, dtype=jnp.float32)
            l0 = jnp.zeros((SQ, 1), dtype=jnp.float32)
            u0 = jnp.zeros((SQ, DH), dtype=jnp.float32)
            m, l, u = lax.fori_loop(0, n_chunks, chunk_body, (m0, l0, u0))
            acc_ref[:, c] = u
            acc_ref[:, M_OFF + h:M_OFF + h + 1] = m
            acc_ref[:, L_OFF + h:L_OFF + h + 1] = l

        for t in range(N_DEV - 1):
            sc = lax.rem(my - t + N_DEV, N_DEV)
            rc = lax.rem(my - t - 1 + N_DEV, N_DEV)
            rdma = pltpu.make_async_remote_copy(
                src_ref=acc_ref.at[pl.ds(sc * R, R)],
                dst_ref=rs_buf.at[t],
                send_sem=rs_send_sems.at[t],
                recv_sem=rs_recv_sems.at[t],
                device_id=(right,),
                device_id_type=pl.DeviceIdType.MESH,
            )
            rdma.start()
            rdma.wait()
            rows = pl.ds(rc * R, R)
            for h in range(HQ):
                c = slice(h * DH, (h + 1) * DH)
                ma = acc_ref[rows, M_OFF + h:M_OFF + h + 1]
                mb = rs_buf[t, :, M_OFF + h:M_OFF + h + 1]
                m_new = jnp.maximum(ma, mb)
                alpha = jnp.exp(ma - m_new)
                beta = jnp.exp(mb - m_new)
                acc_ref[rows, c] = (
                    acc_ref[rows, c] * alpha + rs_buf[t, :, c] * beta
                )
                acc_ref[rows, L_OFF + h:L_OFF + h + 1] = (
                    acc_ref[rows, L_OFF + h:L_OFF + h + 1] * alpha
                    + rs_buf[t, :, L_OFF + h:L_OFF + h + 1] * beta
                )
                acc_ref[rows, M_OFF + h:M_OFF + h + 1] = m_new

        own = lax.rem(my + 1, N_DEV)
        rows = pl.ds(own * R, R)
        for h in range(HQ):
            c = slice(h * DH, (h + 1) * DH)
            acc_ref[rows, c] = (
                acc_ref[rows, c] / acc_ref[rows, L_OFF + h:L_OFF + h + 1]
            )
        out_ref[rows, :] = jnp.dot(
            acc_ref[rows, :D], wo_ref[...], preferred_element_type=jnp.float32
        )

        for t in range(N_DEV - 1):
            g = lax.rem(my + 1 - t + N_DEV, N_DEV)
            rows = pl.ds(g * R, R)
            rdma = pltpu.make_async_remote_copy(
                src_ref=out_ref.at[rows],
                dst_ref=out_ref.at[rows],
                send_sem=ag_send_sems.at[t],
                recv_sem=ag_recv_sems.at[t],
                device_id=(right,),
                device_id_type=pl.DeviceIdType.MESH,
            )
            rdma.start()
            rdma.wait()

    out = pl.pallas_call(
        body,
        out_shape=jax.ShapeDtypeStruct((SQ, D), jnp.float32),
        in_specs=[pl.BlockSpec(memory_space=pltpu.VMEM)] * 5,
        out_specs=pl.BlockSpec(memory_space=pltpu.VMEM),
        scratch_shapes=[
            pltpu.VMEM((SQ, D), jnp.float32),
            pltpu.VMEM((SQ, CW), jnp.float32),
            pltpu.VMEM((N_DEV - 1, R, CW), jnp.float32),
            pltpu.SemaphoreType.DMA((N_DEV - 1,)),
            pltpu.SemaphoreType.DMA((N_DEV - 1,)),
            pltpu.SemaphoreType.DMA((N_DEV - 1,)),
            pltpu.SemaphoreType.DMA((N_DEV - 1,)),
        ],
        compiler_params=pltpu.CompilerParams(collective_id=0),
    )(x2, Wq, Wo, k2, v2)
    return out.reshape(1, SQ, D)


# device time: 79898 ns/iter; 3.1897x vs baseline; 1.4613x over previous
import jax
import jax.numpy as jnp
from jax import lax
from jax.experimental import pallas as pl
from jax.experimental.pallas import tpu as pltpu

N_DEV = 8
SQ = 512
D = 1024
HQ = 8
DH = 128
R = SQ // N_DEV
KV_CHUNK = 1024
SCALE = 0.08838834764831843
CW = D + 128
M_OFF = D
L_OFF = D + 8


def kernel(x, Wq, Wo, K_ext, V_ext):
    skv = K_ext.shape[1]
    x2 = x.reshape(SQ, D)
    k2 = K_ext.reshape(skv, HQ * DH)
    v2 = V_ext.reshape(skv, HQ * DH)

    def body(x_ref, wq_ref, wo_ref, k_ref, v_ref, out_ref,
             q_ref, acc_ref, rs_buf,
             rs_send_sems, rs_recv_sems, ag_send_sems, ag_recv_sems):
        my = lax.axis_index("i")

        barrier_sem = pltpu.get_barrier_semaphore()
        for j in range(1, N_DEV):
            pl.semaphore_signal(
                barrier_sem, inc=1,
                device_id=(lax.rem(my + j, N_DEV),),
                device_id_type=pl.DeviceIdType.MESH,
            )
        pl.semaphore_wait(barrier_sem, N_DEV - 1)

        q_ref[...] = jnp.dot(
            x_ref[...], wq_ref[...], preferred_element_type=jnp.float32
        )

        n_chunks = skv // KV_CHUNK
        for h in range(HQ):
            c = slice(h * DH, (h + 1) * DH)
            qh = q_ref[:, c]

            def chunk_body(ci, carry, c=c, qh=qh):
                m, l, u = carry
                r = pl.ds(ci * KV_CHUNK, KV_CHUNK)
                s = lax.dot_general(
                    qh, k_ref[r, c],
                    (((1,), (1,)), ((), ())),
                    preferred_element_type=jnp.float32,
                ) * SCALE
                mj = jnp.max(s, axis=1, keepdims=True)
                m_new = jnp.maximum(m, mj)
                alpha = jnp.exp(m - m_new)
                p = jnp.exp(s - m_new)
                l = l * alpha + jnp.sum(p, axis=1, keepdims=True)
                u = u * alpha + jnp.dot(
                    p, v_ref[r, c], preferred_element_type=jnp.float32)
                return m_new, l, u

            m0 = jnp.full((SQ, 1), -jnp.inf, dtype=jnp.float32)
            l0 = jnp.zeros((SQ, 1), dtype=jnp.float32)
            u0 = jnp.zeros((SQ, DH), dtype=jnp.float32)
            m, l, u = lax.fori_loop(0, n_chunks, chunk_body, (m0, l0, u0))
            acc_ref[:, c] = u
            acc_ref[:, M_OFF + h:M_OFF + h + 1] = m
            acc_ref[:, L_OFF + h:L_OFF + h + 1] = l

        rs_rdmas = []
        for j in range(1, N_DEV):
            tgt = lax.rem(my + j, N_DEV)
            rdma = pltpu.make_async_remote_copy(
                src_ref=acc_ref.at[pl.ds(tgt * R, R)],
                dst_ref=rs_buf.at[j - 1],
                send_sem=rs_send_sems.at[j - 1],
                recv_sem=rs_recv_sems.at[j - 1],
                device_id=(tgt,),
                device_id_type=pl.DeviceIdType.MESH,
            )
            rdma.start()
            rs_rdmas.append(rdma)

        rows = pl.ds(my * R, R)
        for j in range(1, N_DEV):
            rs_rdmas[j - 1].wait_recv()
            s = j - 1
            for h in range(HQ):
                c = slice(h * DH, (h + 1) * DH)
                ma = acc_ref[rows, M_OFF + h:M_OFF + h + 1]
                mb = rs_buf[s, :, M_OFF + h:M_OFF + h + 1]
                m_new = jnp.maximum(ma, mb)
                alpha = jnp.exp(ma - m_new)
                beta = jnp.exp(mb - m_new)
                acc_ref[rows, c] = (
                    acc_ref[rows, c] * alpha + rs_buf[s, :, c] * beta
                )
                acc_ref[rows, L_OFF + h:L_OFF + h + 1] = (
                    acc_ref[rows, L_OFF + h:L_OFF + h + 1] * alpha
                    + rs_buf[s, :, L_OFF + h:L_OFF + h + 1] * beta
                )
                acc_ref[rows, M_OFF + h:M_OFF + h + 1] = m_new

        for h in range(HQ):
            c = slice(h * DH, (h + 1) * DH)
            acc_ref[rows, c] = (
                acc_ref[rows, c] / acc_ref[rows, L_OFF + h:L_OFF + h + 1]
            )
        out_ref[rows, :] = jnp.dot(
            acc_ref[rows, :D], wo_ref[...], preferred_element_type=jnp.float32
        )

        ag_rdmas = []
        for j in range(1, N_DEV):
            tgt = lax.rem(my + j, N_DEV)
            rdma = pltpu.make_async_remote_copy(
                src_ref=out_ref.at[rows],
                dst_ref=out_ref.at[rows],
                send_sem=ag_send_sems.at[j - 1],
                recv_sem=ag_recv_sems.at[j - 1],
                device_id=(tgt,),
                device_id_type=pl.DeviceIdType.MESH,
            )
            rdma.start()
            ag_rdmas.append(rdma)

        for rdma in rs_rdmas:
            rdma.wait_send()
        for rdma in ag_rdmas:
            rdma.wait_recv()
        for rdma in ag_rdmas:
            rdma.wait_send()

    out = pl.pallas_call(
        body,
        out_shape=jax.ShapeDtypeStruct((SQ, D), jnp.float32),
        in_specs=[pl.BlockSpec(memory_space=pltpu.VMEM)] * 5,
        out_specs=pl.BlockSpec(memory_space=pltpu.VMEM),
        scratch_shapes=[
            pltpu.VMEM((SQ, D), jnp.float32),
            pltpu.VMEM((SQ, CW), jnp.float32),
            pltpu.VMEM((N_DEV - 1, R, CW), jnp.float32),
            pltpu.SemaphoreType.DMA((N_DEV - 1,)),
            pltpu.SemaphoreType.DMA((N_DEV - 1,)),
            pltpu.SemaphoreType.DMA((N_DEV - 1,)),
            pltpu.SemaphoreType.DMA((N_DEV - 1,)),
        ],
        compiler_params=pltpu.CompilerParams(collective_id=0),
    )(x2, Wq, Wo, k2, v2)
    return out.reshape(1, SQ, D)


# device time: 75296 ns/iter; 3.3847x vs baseline; 1.0611x over previous
import jax
import jax.numpy as jnp
from jax import lax
from jax.experimental import pallas as pl
from jax.experimental.pallas import tpu as pltpu

N_DEV = 8
SQ = 512
D = 1024
HQ = 8
DH = 128
R = SQ // N_DEV
KV_CHUNK = 1024
SCALE = 0.08838834764831843
CW = D + 128
M_OFF = D
L_OFF = D + 8


def kernel(x, Wq, Wo, K_ext, V_ext):
    skv = K_ext.shape[1]
    x2 = x.reshape(SQ, D).astype(jnp.bfloat16)
    wq_bf = Wq.astype(jnp.bfloat16)
    k2 = K_ext.reshape(skv, HQ * DH).astype(jnp.bfloat16)
    v2 = V_ext.reshape(skv, HQ * DH).astype(jnp.bfloat16)

    def body(x_ref, wq_ref, wo_ref, k_ref, v_ref, out_ref,
             q_ref, acc_ref, rs_buf,
             rs_send_sems, rs_recv_sems, ag_send_sems, ag_recv_sems):
        my = lax.axis_index("i")

        barrier_sem = pltpu.get_barrier_semaphore()
        for j in range(1, N_DEV):
            pl.semaphore_signal(
                barrier_sem, inc=1,
                device_id=(lax.rem(my + j, N_DEV),),
                device_id_type=pl.DeviceIdType.MESH,
            )
        pl.semaphore_wait(barrier_sem, N_DEV - 1)

        q_ref[...] = jnp.dot(
            x_ref[...], wq_ref[...], preferred_element_type=jnp.float32
        ).astype(jnp.bfloat16)

        n_chunks = skv // KV_CHUNK
        for h in range(HQ):
            c = slice(h * DH, (h + 1) * DH)
            qh = q_ref[:, c]

            def chunk_body(ci, carry, c=c, qh=qh):
                m, l, u = carry
                r = pl.ds(ci * KV_CHUNK, KV_CHUNK)
                s = lax.dot_general(
                    qh, k_ref[r, c],
                    (((1,), (1,)), ((), ())),
                    preferred_element_type=jnp.float32,
                ) * SCALE
                mj = jnp.max(s, axis=1, keepdims=True)
                m_new = jnp.maximum(m, mj)
                alpha = jnp.exp(m - m_new)
                p = jnp.exp(s - m_new)
                l = l * alpha + jnp.sum(p, axis=1, keepdims=True)
                u = u * alpha + jnp.dot(
                    p.astype(jnp.bfloat16), v_ref[r, c],
                    preferred_element_type=jnp.float32)
                return m_new, l, u

            m0 = jnp.full((SQ, 1), -jnp.inf, dtype=jnp.float32)
            l0 = jnp.zeros((SQ, 1), dtype=jnp.float32)
            u0 = jnp.zeros((SQ, DH), dtype=jnp.float32)
            m, l, u = lax.fori_loop(0, n_chunks, chunk_body, (m0, l0, u0))
            acc_ref[:, c] = u
            acc_ref[:, M_OFF + h:M_OFF + h + 1] = m
            acc_ref[:, L_OFF + h:L_OFF + h + 1] = l

        rs_rdmas = []
        for j in range(1, N_DEV):
            tgt = lax.rem(my + j, N_DEV)
            rdma = pltpu.make_async_remote_copy(
                src_ref=acc_ref.at[pl.ds(tgt * R, R)],
                dst_ref=rs_buf.at[j - 1],
                send_sem=rs_send_sems.at[j - 1],
                recv_sem=rs_recv_sems.at[j - 1],
                device_id=(tgt,),
                device_id_type=pl.DeviceIdType.MESH,
            )
            rdma.start()
            rs_rdmas.append(rdma)

        rows = pl.ds(my * R, R)
        for j in range(1, N_DEV):
            rs_rdmas[j - 1].wait_recv()
            s = j - 1
            for h in range(HQ):
                c = slice(h * DH, (h + 1) * DH)
                ma = acc_ref[rows, M_OFF + h:M_OFF + h + 1]
                mb = rs_buf[s, :, M_OFF + h:M_OFF + h + 1]
                m_new = jnp.maximum(ma, mb)
                alpha = jnp.exp(ma - m_new)
                beta = jnp.exp(mb - m_new)
                acc_ref[rows, c] = (
                    acc_ref[rows, c] * alpha + rs_buf[s, :, c] * beta
                )
                acc_ref[rows, L_OFF + h:L_OFF + h + 1] = (
                    acc_ref[rows, L_OFF + h:L_OFF + h + 1] * alpha
                    + rs_buf[s, :, L_OFF + h:L_OFF + h + 1] * beta
                )
                acc_ref[rows, M_OFF + h:M_OFF + h + 1] = m_new

        for h in range(HQ):
            c = slice(h * DH, (h + 1) * DH)
            acc_ref[rows, c] = (
                acc_ref[rows, c] / acc_ref[rows, L_OFF + h:L_OFF + h + 1]
            )
        out_ref[rows, :] = jnp.dot(
            acc_ref[rows, :D], wo_ref[...], preferred_element_type=jnp.float32
        )

        ag_rdmas = []
        for j in range(1, N_DEV):
            tgt = lax.rem(my + j, N_DEV)
            rdma = pltpu.make_async_remote_copy(
                src_ref=out_ref.at[rows],
                dst_ref=out_ref.at[rows],
                send_sem=ag_send_sems.at[j - 1],
                recv_sem=ag_recv_sems.at[j - 1],
                device_id=(tgt,),
                device_id_type=pl.DeviceIdType.MESH,
            )
            rdma.start()
            ag_rdmas.append(rdma)

        for rdma in rs_rdmas:
            rdma.wait_send()
        for rdma in ag_rdmas:
            rdma.wait_recv()
        for rdma in ag_rdmas:
            rdma.wait_send()

    out = pl.pallas_call(
        body,
        out_shape=jax.ShapeDtypeStruct((SQ, D), jnp.float32),
        in_specs=[pl.BlockSpec(memory_space=pltpu.VMEM)] * 5,
        out_specs=pl.BlockSpec(memory_space=pltpu.VMEM),
        scratch_shapes=[
            pltpu.VMEM((SQ, D), jnp.bfloat16),
            pltpu.VMEM((SQ, CW), jnp.float32),
            pltpu.VMEM((N_DEV - 1, R, CW), jnp.float32),
            pltpu.SemaphoreType.DMA((N_DEV - 1,)),
            pltpu.SemaphoreType.DMA((N_DEV - 1,)),
            pltpu.SemaphoreType.DMA((N_DEV - 1,)),
            pltpu.SemaphoreType.DMA((N_DEV - 1,)),
        ],
        compiler_params=pltpu.CompilerParams(collective_id=0),
    )(x2, wq_bf, Wo, k2, v2)
    return out.reshape(1, SQ, D)


# device time: 60656 ns/iter; 4.2016x vs baseline; 1.2414x over previous
import os

import jax
import jax.numpy as jnp
from jax import lax
from jax.experimental import pallas as pl
from jax.experimental.pallas import tpu as pltpu

_NOCOMM = bool(int(os.environ.get("NOCOMM", "0")))

N_DEV = 8
SQ = 512
D = 1024
HQ = 8
DH = 128
R = SQ // N_DEV
KV_CHUNK = 1024
SCALE = 0.08838834764831843
CW = D + 128
M_OFF = D
L_OFF = D + 8


def kernel(x, Wq, Wo, K_ext, V_ext):
    skv = K_ext.shape[1]
    x2 = x.reshape(SQ, D).astype(jnp.bfloat16)
    wq_bf = Wq.astype(jnp.bfloat16)
    k2 = K_ext.reshape(skv, HQ * DH).astype(jnp.bfloat16)
    v2 = V_ext.reshape(skv, HQ * DH).astype(jnp.bfloat16)

    def body(x_ref, wq_ref, wo_ref, k_ref, v_ref, out_ref,
             q_ref, acc_ref, rs_ostage, rs_obuf, rs_mlbuf, ag_stage, ag_buf,
             rs_osend_sems, rs_orecv_sems, rs_mlsend_sems, rs_mlrecv_sems,
             ag_send_sems, ag_recv_sems):
        my = lax.axis_index("i")

        if not _NOCOMM:
            barrier_sem = pltpu.get_barrier_semaphore()
            for j in range(1, N_DEV):
                pl.semaphore_signal(
                    barrier_sem, inc=1,
                    device_id=(lax.rem(my + j, N_DEV),),
                    device_id_type=pl.DeviceIdType.MESH,
                )
            pl.semaphore_wait(barrier_sem, N_DEV - 1)

        q_ref[...] = jnp.dot(
            x_ref[...], wq_ref[...], preferred_element_type=jnp.float32
        ).astype(jnp.bfloat16)

        n_chunks = skv // KV_CHUNK
        for h in range(HQ):
            c = slice(h * DH, (h + 1) * DH)
            qh = q_ref[:, c]

            def chunk_body(ci, carry, c=c, qh=qh):
                m, l, u = carry
                r = pl.ds(ci * KV_CHUNK, KV_CHUNK)
                s = lax.dot_general(
                    qh, k_ref[r, c],
                    (((1,), (1,)), ((), ())),
                    preferred_element_type=jnp.float32,
                ) * SCALE
                mj = jnp.max(s, axis=1, keepdims=True)
                m_new = jnp.maximum(m, mj)
                alpha = jnp.exp(m - m_new)
                p = jnp.exp(s - m_new)
                l = l * alpha + jnp.sum(p, axis=1, keepdims=True)
                u = u * alpha + jnp.dot(
                    p.astype(jnp.bfloat16), v_ref[r, c],
                    preferred_element_type=jnp.float32)
                return m_new, l, u

            m0 = jnp.full((SQ, 1), -jnp.inf, dtype=jnp.float32)
            l0 = jnp.zeros((SQ, 1), dtype=jnp.float32)
            u0 = jnp.zeros((SQ, DH), dtype=jnp.float32)
            m, l, u = lax.fori_loop(0, n_chunks, chunk_body, (m0, l0, u0))
            acc_ref[:, c] = u
            acc_ref[:, M_OFF + h:M_OFF + h + 1] = m
            acc_ref[:, L_OFF + h:L_OFF + h + 1] = l

        rs_o_rdmas = []
        rs_ml_rdmas = []
        for j in range(1, N_DEV) if not _NOCOMM else ():
            tgt = lax.rem(my + j, N_DEV)
            trows = pl.ds(tgt * R, R)
            rs_ostage[j - 1, :, :] = acc_ref[trows, :D].astype(jnp.bfloat16)
            o_rdma = pltpu.make_async_remote_copy(
                src_ref=rs_ostage.at[j - 1],
                dst_ref=rs_obuf.at[j - 1],
                send_sem=rs_osend_sems.at[j - 1],
                recv_sem=rs_orecv_sems.at[j - 1],
                device_id=(tgt,),
                device_id_type=pl.DeviceIdType.MESH,
            )
            o_rdma.start()
            rs_o_rdmas.append(o_rdma)
            ml_rdma = pltpu.make_async_remote_copy(
                src_ref=acc_ref.at[trows, pl.ds(M_OFF, DH)],
                dst_ref=rs_mlbuf.at[j - 1],
                send_sem=rs_mlsend_sems.at[j - 1],
                recv_sem=rs_mlrecv_sems.at[j - 1],
                device_id=(tgt,),
                device_id_type=pl.DeviceIdType.MESH,
            )
            ml_rdma.start()
            rs_ml_rdmas.append(ml_rdma)

        rows = pl.ds(my * R, R)
        for j in range(1, N_DEV) if not _NOCOMM else ():
            s = j - 1
            rs_o_rdmas[s].wait_recv()
            rs_ml_rdmas[s].wait_recv()
            o_in = rs_obuf[s].astype(jnp.float32)
            for h in range(HQ):
                c = slice(h * DH, (h + 1) * DH)
                ma = acc_ref[rows, M_OFF + h:M_OFF + h + 1]
                mb = rs_mlbuf[s, :, h:h + 1]
                m_new = jnp.maximum(ma, mb)
                alpha = jnp.exp(ma - m_new)
                beta = jnp.exp(mb - m_new)
                acc_ref[rows, c] = (
                    acc_ref[rows, c] * alpha + o_in[:, c] * beta
                )
                acc_ref[rows, L_OFF + h:L_OFF + h + 1] = (
                    acc_ref[rows, L_OFF + h:L_OFF + h + 1] * alpha
                    + rs_mlbuf[s, :, HQ + h:HQ + h + 1] * beta
                )
                acc_ref[rows, M_OFF + h:M_OFF + h + 1] = m_new

        for h in range(HQ):
            c = slice(h * DH, (h + 1) * DH)
            acc_ref[rows, c] = (
                acc_ref[rows, c] / acc_ref[rows, L_OFF + h:L_OFF + h + 1]
            )
        out_ref[rows, :] = jnp.dot(
            acc_ref[rows, :D], wo_ref[...], preferred_element_type=jnp.float32
        )

        ag_rdmas = []
        if not _NOCOMM:
            ag_stage[...] = out_ref[rows, :].astype(jnp.bfloat16)
        for j in range(1, N_DEV) if not _NOCOMM else ():
            tgt = lax.rem(my + j, N_DEV)
            rdma = pltpu.make_async_remote_copy(
                src_ref=ag_stage,
                dst_ref=ag_buf.at[j - 1],
                send_sem=ag_send_sems.at[j - 1],
                recv_sem=ag_recv_sems.at[j - 1],
                device_id=(tgt,),
                device_id_type=pl.DeviceIdType.MESH,
            )
            rdma.start()
            ag_rdmas.append(rdma)

        for rdma in rs_o_rdmas:
            rdma.wait_send()
        for rdma in rs_ml_rdmas:
            rdma.wait_send()
        for j in range(1, N_DEV) if not _NOCOMM else ():
            s = j - 1
            ag_rdmas[s].wait_recv()
            src_dev = lax.rem(my - j + N_DEV, N_DEV)
            out_ref[pl.ds(src_dev * R, R), :] = (
                ag_buf[s].astype(jnp.float32)
            )
        for rdma in ag_rdmas:
            rdma.wait_send()

    out = pl.pallas_call(
        body,
        out_shape=jax.ShapeDtypeStruct((SQ, D), jnp.float32),
        in_specs=[pl.BlockSpec(memory_space=pltpu.VMEM)] * 5,
        out_specs=pl.BlockSpec(memory_space=pltpu.VMEM),
        scratch_shapes=[
            pltpu.VMEM((SQ, D), jnp.bfloat16),
            pltpu.VMEM((SQ, CW), jnp.float32),
            pltpu.VMEM((N_DEV - 1, R, D), jnp.bfloat16),
            pltpu.VMEM((N_DEV - 1, R, D), jnp.bfloat16),
            pltpu.VMEM((N_DEV - 1, R, DH), jnp.float32),
            pltpu.VMEM((R, D), jnp.bfloat16),
            pltpu.VMEM((N_DEV - 1, R, D), jnp.bfloat16),
            pltpu.SemaphoreType.DMA((N_DEV - 1,)),
            pltpu.SemaphoreType.DMA((N_DEV - 1,)),
            pltpu.SemaphoreType.DMA((N_DEV - 1,)),
            pltpu.SemaphoreType.DMA((N_DEV - 1,)),
            pltpu.SemaphoreType.DMA((N_DEV - 1,)),
            pltpu.SemaphoreType.DMA((N_DEV - 1,)),
        ],
        compiler_params=(
            pltpu.CompilerParams()
            if _NOCOMM
            else pltpu.CompilerParams(collective_id=0)
        ),
    )(x2, wq_bf, Wo, k2, v2)
    return out.reshape(1, SQ, D)
